# Initial kernel scaffold; baseline (speedup 1.0000x reference)
import jax
import jax.numpy as jnp
from jax.experimental import pallas as pl


def kernel(node_feats, batch, ref_weight):
    raise NotImplementedError("write your pallas kernel here")


# speedup vs baseline: 21.4909x; 21.4909x over previous
import functools

import jax
import jax.numpy as jnp
from jax import lax
from jax.experimental import pallas as pl
from jax.experimental.pallas import tpu as pltpu
from jax.experimental.pallas import tpu_sc as plsc

N_GRAPHS = 1024
LANES = 16
N_SUBCORES = 16


@functools.lru_cache(maxsize=None)
def _build_sc_call(n_pad, n_species_pad, n_graphs):
    chunk = n_pad // N_SUBCORES
    n_vregs = chunk // LANES
    acc_len = ((n_graphs + 1 + LANES - 1) // LANES) * LANES
    span = 128
    n_comb = n_graphs // span
    span_vregs = span // LANES

    mesh = plsc.VectorSubcoreMesh(
        core_axis_name="c", subcore_axis_name="s", num_cores=1
    )

    @functools.partial(
        pl.kernel,
        out_type=jax.ShapeDtypeStruct((n_graphs,), jnp.float32),
        mesh=mesh,
        compiler_params=pltpu.CompilerParams(needs_layout_passes=False),
        scratch_types=[
            pltpu.VMEM((chunk,), jnp.int32),
            pltpu.VMEM((chunk + LANES,), jnp.int32),
            pltpu.VMEM((n_species_pad,), jnp.float32),
            pltpu.VMEM((acc_len,), jnp.float32),
            pltpu.VMEM((LANES,), jnp.int32),
            pltpu.VMEM((N_SUBCORES, LANES), jnp.int32),
            pltpu.VMEM((N_SUBCORES, 128), jnp.float32),
            pltpu.VMEM((128,), jnp.float32),
            pltpu.VMEM_SHARED((N_SUBCORES, LANES), jnp.int32),
            pltpu.VMEM_SHARED((N_SUBCORES, acc_len), jnp.float32),
        ],
    )
    def sc_call(idx_hbm, bat_hbm, w_hbm, out_hbm,
                idx_v, bat_v, w_v, acc_v, min_v, gmin_v, comb_v, out_v,
                mins_sh, acc_sh):
        tid = lax.axis_index("s")
        base = tid * chunk
        pltpu.sync_copy(idx_hbm.at[pl.ds(base, chunk)], idx_v)
        pltpu.sync_copy(bat_hbm.at[pl.ds(base, chunk + LANES)], bat_v)
        pltpu.sync_copy(w_hbm, w_v)

        def min_body(j, m):
            return jnp.minimum(m, idx_v[pl.ds(j * LANES, LANES)])

        m = lax.fori_loop(
            0, n_vregs, min_body, jnp.full((LANES,), 2**30, jnp.int32)
        )
        min_v[...] = m
        pltpu.sync_copy(min_v, mins_sh.at[tid])

        def zero_body(j, carry):
            acc_v[pl.ds(j * LANES, LANES)] = jnp.zeros((LANES,), jnp.float32)
            return carry

        lax.fori_loop(0, acc_len // LANES, zero_body, 0)
        plsc.subcore_barrier()

        pltpu.sync_copy(mins_sh, gmin_v)

        def gmin_body(k, mm):
            return jnp.minimum(mm, gmin_v[k, :])

        mm = lax.fori_loop(
            0, N_SUBCORES, gmin_body, jnp.full((LANES,), 2**30, jnp.int32)
        )
        gmin = mm[0]
        for k in range(1, LANES):
            gmin = jnp.minimum(gmin, mm[k])
        shift = jnp.where(gmin >= 1, jnp.int32(1), jnp.int32(0))

        lane = lax.iota(jnp.int32, LANES)
        last_lane = lane == (LANES - 1)
        not_last = lane < (LANES - 1)

        def main_body(j, carry):
            b = bat_v[pl.ds(j * LANES, LANES)]
            b2 = bat_v[pl.ds(j * LANES + 1, LANES)]
            i = idx_v[pl.ds(j * LANES, LANES)]
            i = jnp.maximum(i - shift, 0)
            v = plsc.load_gather(w_v, [i])
            cs = plsc.cumsum(v)
            diff = b != b2
            plsc.addupdate_scatter(acc_v, [b], cs, mask=diff | last_lane)
            plsc.addupdate_scatter(acc_v, [b2], -cs, mask=diff & not_last)
            return carry

        lax.fori_loop(0, n_vregs, main_body, 0)

        pltpu.sync_copy(acc_v, acc_sh.at[tid])
        plsc.subcore_barrier()

        @pl.when(tid < n_comb)
        def _():
            pltpu.sync_copy(acc_sh.at[:, pl.ds(tid * span, span)], comb_v)

            def comb_body(k, carry):
                return tuple(
                    carry[c] + comb_v[k, pl.ds(c * LANES, LANES)]
                    for c in range(span_vregs)
                )

            ss = lax.fori_loop(
                0, N_SUBCORES, comb_body,
                tuple(jnp.zeros((LANES,), jnp.float32)
                      for _ in range(span_vregs)),
            )
            for c in range(span_vregs):
                out_v[pl.ds(c * LANES, LANES)] = ss[c]
            pltpu.sync_copy(out_v, out_hbm.at[pl.ds(tid * span, span)])

    return sc_call


@jax.jit
def kernel(node_feats, batch, ref_weight):
    n_nodes = node_feats.shape[0]
    n_species, out_dim = ref_weight.shape
    n_pad = ((n_nodes + N_SUBCORES * 8 - 1) // (N_SUBCORES * 8)) * (N_SUBCORES * 8)
    n_species_pad = ((n_species + LANES - 1) // LANES) * LANES

    idx = node_feats[:, 0].astype(jnp.int32)
    idx_p = jnp.pad(idx, (0, n_pad - n_nodes), constant_values=n_species - 1)
    bat_p = jnp.pad(
        batch.astype(jnp.int32),
        (0, n_pad + LANES - n_nodes),
        constant_values=N_GRAPHS,
    )
    w_p = jnp.pad(ref_weight[:, 0], (0, n_species_pad - n_species))

    sc_call = _build_sc_call(n_pad, n_species_pad, N_GRAPHS)
    out = sc_call(idx_p, bat_p, w_p)
    return out.reshape(N_GRAPHS, out_dim)


# speedup vs baseline: 23.0207x; 1.0712x over previous
import functools

import jax
import jax.numpy as jnp
from jax import lax
from jax.experimental import pallas as pl
from jax.experimental.pallas import tpu as pltpu
from jax.experimental.pallas import tpu_sc as plsc

N_GRAPHS = 1024
LANES = 16
N_SUBCORES = 16


@functools.lru_cache(maxsize=None)
def _build_sc_call(n_nodes, n_species, n_graphs):
    chunk = -(-n_nodes // (N_SUBCORES * 8)) * 8
    n_vregs = -(-chunk // LANES)
    chunk = n_vregs * LANES
    last = N_SUBCORES - 1
    valid_last = n_nodes - last * chunk
    fill_from = valid_last - valid_last % LANES
    acc_len = -(-(n_graphs + 1) // LANES) * LANES
    span = 128
    n_comb = n_graphs // span
    span_vregs = span // LANES

    mesh = plsc.VectorSubcoreMesh(
        core_axis_name="c", subcore_axis_name="s", num_cores=1
    )

    @functools.partial(
        pl.kernel,
        out_type=jax.ShapeDtypeStruct((n_graphs,), jnp.float32),
        mesh=mesh,
        compiler_params=pltpu.CompilerParams(needs_layout_passes=False),
        scratch_types=[
            pltpu.VMEM((chunk,), jnp.int32),
            pltpu.VMEM((chunk + LANES,), jnp.int32),
            pltpu.VMEM((-(-n_species // LANES) * LANES,), jnp.float32),
            pltpu.VMEM((acc_len,), jnp.float32),
            pltpu.VMEM((LANES,), jnp.int32),
            pltpu.VMEM((N_SUBCORES, LANES), jnp.int32),
            pltpu.VMEM((N_SUBCORES, span), jnp.float32),
            pltpu.VMEM((span,), jnp.float32),
            pltpu.VMEM_SHARED((N_SUBCORES, LANES), jnp.int32),
            pltpu.VMEM_SHARED((N_SUBCORES, acc_len), jnp.float32),
            pltpu.SemaphoreType.DMA,
            pltpu.SemaphoreType.DMA,
            pltpu.SemaphoreType.DMA,
        ],
    )
    def sc_call(idx_hbm, bat_hbm, w_hbm, out_hbm,
                idx_v, bat_v, w_v, acc_v, min_v, gmin_v, comb_v, out_v,
                mins_sh, acc_sh, sem1, sem2, sem3):
        tid = lax.axis_index("s")
        base = tid * chunk

        @pl.when(tid < last)
        def _():
            pltpu.sync_copy(idx_hbm.at[pl.ds(base, chunk)], idx_v)
            pltpu.sync_copy(bat_hbm.at[pl.ds(base, chunk + LANES)], bat_v)
            pltpu.sync_copy(w_hbm, w_v.at[pl.ds(0, n_species)])

        @pl.when(tid == last)
        def _():
            pltpu.sync_copy(
                idx_hbm.at[pl.ds(base, valid_last)],
                idx_v.at[pl.ds(0, valid_last)],
            )
            pltpu.sync_copy(
                bat_hbm.at[pl.ds(base, valid_last)],
                bat_v.at[pl.ds(0, valid_last)],
            )
            pltpu.sync_copy(w_hbm, w_v.at[pl.ds(0, n_species)])
            pad_i = jnp.full((LANES,), n_species - 1, jnp.int32)
            pad_b = jnp.full((LANES,), n_graphs, jnp.int32)
            for j in range(fill_from, chunk + LANES, LANES):
                if j + LANES > valid_last:
                    lo = lax.iota(jnp.int32, LANES) + (j - valid_last)
                    keep = lo < 0
                    if j < valid_last:
                        ii = idx_v[pl.ds(j, LANES)]
                        bb = bat_v[pl.ds(j, LANES)]
                        idx_v[pl.ds(j, LANES)] = jnp.where(keep, ii, pad_i)
                        bat_v[pl.ds(j, LANES)] = jnp.where(keep, bb, pad_b)
                    else:
                        if j < chunk:
                            idx_v[pl.ds(j, LANES)] = pad_i
                        bat_v[pl.ds(j, LANES)] = pad_b

        def min_body(j, m):
            return jnp.minimum(m, idx_v[pl.ds(j * LANES, LANES)])

        m = lax.fori_loop(
            0, n_vregs, min_body, jnp.full((LANES,), 2**30, jnp.int32),
        )
        min_v[...] = m
        pltpu.sync_copy(min_v, mins_sh.at[tid])

        def zero_body(j, carry):
            acc_v[pl.ds(j * LANES, LANES)] = jnp.zeros((LANES,), jnp.float32)
            return carry

        lax.fori_loop(0, acc_len // LANES, zero_body, 0)
        plsc.subcore_barrier()

        pltpu.sync_copy(mins_sh, gmin_v)

        def gmin_body(k, mm):
            return jnp.minimum(mm, gmin_v[k, :])

        mm = lax.fori_loop(
            0, N_SUBCORES, gmin_body, jnp.full((LANES,), 2**30, jnp.int32),
        )
        gmin = mm[0]
        for k in range(1, LANES):
            gmin = jnp.minimum(gmin, mm[k])
        shift = jnp.where(gmin >= 1, jnp.int32(1), jnp.int32(0))

        lane = lax.iota(jnp.int32, LANES)
        last_lane = lane == (LANES - 1)
        not_last = lane < (LANES - 1)

        def main_body(j, carry):
            b = bat_v[pl.ds(j * LANES, LANES)]
            b2 = bat_v[pl.ds(j * LANES + 1, LANES)]
            i = idx_v[pl.ds(j * LANES, LANES)]
            i = jnp.maximum(i - shift, 0)
            v = plsc.load_gather(w_v, [i])
            cs = plsc.cumsum(v)
            diff = b != b2
            plsc.addupdate_scatter(acc_v, [b], cs, mask=diff | last_lane)
            plsc.addupdate_scatter(acc_v, [b2], -cs, mask=diff & not_last)
            return carry

        lax.fori_loop(0, n_vregs, main_body, 0)

        pltpu.sync_copy(acc_v, acc_sh.at[tid])
        plsc.subcore_barrier()

        @pl.when(tid < n_comb)
        def _():
            pltpu.sync_copy(acc_sh.at[:, pl.ds(tid * span, span)], comb_v)

            def comb_body(k, carry):
                return tuple(
                    carry[c] + comb_v[k, pl.ds(c * LANES, LANES)]
                    for c in range(span_vregs)
                )

            ss = lax.fori_loop(
                0, N_SUBCORES, comb_body,
                tuple(jnp.zeros((LANES,), jnp.float32)
                      for _ in range(span_vregs)),
            )
            for c in range(span_vregs):
                out_v[pl.ds(c * LANES, LANES)] = ss[c]
            pltpu.sync_copy(out_v, out_hbm.at[pl.ds(tid * span, span)])

    return sc_call


@jax.jit
def kernel(node_feats, batch, ref_weight):
    n_nodes = node_feats.shape[0]
    n_species, out_dim = ref_weight.shape
    idx = node_feats[:, 0].astype(jnp.int32)
    sc_call = _build_sc_call(n_nodes, n_species, N_GRAPHS)
    out = sc_call(idx, batch.astype(jnp.int32), ref_weight[:, 0])
    return out.reshape(N_GRAPHS, out_dim)
